# Initial kernel scaffold; baseline (speedup 1.0000x reference)
import jax
import jax.numpy as jnp
from jax.experimental import pallas as pl


def kernel(logits, shape):
    raise NotImplementedError("write your pallas kernel here")


# speedup vs baseline: 1.0890x; 1.0890x over previous
import jax
import jax.numpy as jnp
from jax import lax
from jax.experimental import pallas as pl

K = 32
B = 128
N = 32768
R = 8


def _topk_mask_body(x_ref, o_ref):
    x = x_ref[...]
    xb = lax.bitcast_convert_type(x, jnp.int32)
    flip = (xb >> 31) | jnp.int32(-2147483648)
    u = lax.bitcast_convert_type(xb ^ flip, jnp.uint32)

    kk = jnp.int32(K)

    def step(i, t):
        shift = jnp.uint32(31) - i.astype(jnp.uint32)
        cand = t | lax.shift_left(jnp.uint32(1), shift)
        cnt = jnp.sum((u >= cand).astype(jnp.int32), axis=1, keepdims=True)
        return jnp.where(cnt >= kk, cand, t)

    t0 = jnp.zeros((R, 1), jnp.uint32)
    thr = lax.fori_loop(0, 32, step, t0)

    gt = u > thr
    tie = u == thr
    cnt_gt = jnp.sum(gt.astype(jnp.int32), axis=1, keepdims=True)
    m = kk - cnt_gt

    idx = lax.broadcasted_iota(jnp.int32, (R, N), 1)
    tie_i = tie.astype(jnp.int32)

    def istep(i, j):
        cand = j | lax.shift_left(jnp.int32(1), jnp.int32(14) - i)
        h = jnp.sum(jnp.where(idx < cand, tie_i, 0), axis=1, keepdims=True)
        return jnp.where(h < m, cand, j)

    jstar = lax.fori_loop(0, 15, istep, jnp.zeros((R, 1), jnp.int32))

    sel = jnp.logical_and(tie, idx <= jstar)
    mask = jnp.logical_or(gt, sel)
    o_ref[...] = mask.astype(jnp.float32)


def kernel(logits, shape):
    del shape
    x = logits[..., 0]
    out = pl.pallas_call(
        _topk_mask_body,
        grid=(B // R,),
        in_specs=[pl.BlockSpec((R, N), lambda i: (i, 0))],
        out_specs=pl.BlockSpec((R, N), lambda i: (i, 0)),
        out_shape=jax.ShapeDtypeStruct((B, N), jnp.float32),
    )(x)
    return out[..., None]


# speedup vs baseline: 1.4327x; 1.3156x over previous
import jax
import jax.numpy as jnp
from jax import lax
from jax.experimental import pallas as pl

K = 32
B = 128
N = 32768
R = 8


def _topk_mask_body(x_ref, o_ref):
    x = x_ref[...]
    xb = lax.bitcast_convert_type(x, jnp.int32)
    flip = (xb >> 31) | jnp.int32(-2147483648)
    u = lax.bitcast_convert_type(xb ^ flip, jnp.uint32)

    kk = jnp.int32(K)

    def step(i, t):
        shift = jnp.uint32(31) - i.astype(jnp.uint32)
        cand = t | lax.shift_left(jnp.uint32(1), shift)
        cnt = jnp.sum((u >= cand).astype(jnp.int32), axis=1, keepdims=True)
        return jnp.where(cnt >= kk, cand, t)

    t0 = jnp.zeros((R, 1), jnp.uint32)
    thr = lax.fori_loop(0, 32, step, t0)

    gt = u > thr
    tie = u == thr
    cnt_gt = jnp.sum(gt.astype(jnp.int32), axis=1, keepdims=True)
    tie_i = tie.astype(jnp.int32)
    t_cnt = jnp.sum(tie_i, axis=1, keepdims=True)
    m = kk - cnt_gt

    idx = lax.broadcasted_iota(jnp.int32, (R, N), 1)

    def tie_search():
        def istep(i, j):
            cand = j | lax.shift_left(jnp.int32(1), jnp.int32(14) - i)
            h = jnp.sum(jnp.where(idx < cand, tie_i, 0), axis=1, keepdims=True)
            return jnp.where(h < m, cand, j)

        return lax.fori_loop(0, 15, istep, jnp.zeros((R, 1), jnp.int32))

    need = jnp.any(t_cnt > m)
    jstar = lax.cond(need, tie_search, lambda: jnp.full((R, 1), N, jnp.int32))

    sel = jnp.logical_and(tie, idx <= jstar)
    mask = jnp.logical_or(gt, sel)
    o_ref[...] = mask.astype(jnp.float32)


def kernel(logits, shape):
    del shape
    x = logits[..., 0]
    out = pl.pallas_call(
        _topk_mask_body,
        grid=(B // R,),
        in_specs=[pl.BlockSpec((R, N), lambda i: (i, 0))],
        out_specs=pl.BlockSpec((R, N), lambda i: (i, 0)),
        out_shape=jax.ShapeDtypeStruct((B, N), jnp.float32),
    )(x)
    return out[..., None]


# speedup vs baseline: 1.5019x; 1.0483x over previous
import functools

import jax
import jax.numpy as jnp
from jax import lax
from jax.experimental import pallas as pl
from jax.experimental.pallas import tpu as pltpu
from jax.experimental.pallas import tpu_sc as plsc

K = 32
B = 128
N = 32768
NC = 2
NS = 16
L = 16
NW = NC * NS
RPW = B // NW
NV = N // L
GV = 4
NG = NV // GV
SCAP = 16384
CAPW = SCAP - L
BIG = 2147483647


def _ord_u32(v):
    xb = lax.bitcast_convert_type(v, jnp.int32)
    flip = (xb >> 31) | jnp.int32(-2147483648)
    return lax.bitcast_convert_type(xb ^ flip, jnp.uint32)


def _red16(scr, v, pad, op):
    scr[pl.ds(16, L)] = pad
    s = v
    for sh in (8, 4, 2, 1):
        scr[pl.ds(0, L)] = s
        s = op(s, scr[pl.ds(sh, L)])
    return s[0]


def _sc_body(x_hbm, out_hbm, row_v, outbuf, skey, sidx, tbuf, scr_f, scr_i):
    wid = lax.axis_index("s") * NC + lax.axis_index("c")
    iota16 = lax.broadcasted_iota(jnp.int32, (L,), 0)
    zeros16 = jnp.zeros((L,), jnp.float32)
    ones16 = jnp.ones((L,), jnp.float32)
    izeros16 = jnp.zeros((L,), jnp.int32)
    ione16 = izeros16 + 1
    inf16 = jnp.full((L,), jnp.inf, jnp.float32)
    kk = jnp.int32(K)

    def zinit(i, c):
        outbuf[pl.ds(i * L, L)] = zeros16
        return c

    lax.fori_loop(0, NV, zinit, 0)

    def row_body(j, carry):
        r = wid * RPW + j
        pltpu.sync_copy(x_hbm.at[r], row_v)

        def amax(i, h):
            return jnp.maximum(h, row_v[pl.ds(i * L, L)])

        qn = NV // 4
        khs = []
        for qq in range(4):
            h = lax.fori_loop(qq * qn, (qq + 1) * qn, amax,
                              jnp.full((L,), -jnp.inf))
            khs.append(_ord_u32(h))

        def lb_iter(i, t):
            cand = t | lax.shift_left(
                jnp.uint32(1), (jnp.int32(31) - i).astype(jnp.uint32))
            c = izeros16
            for kh in khs:
                c = c + jnp.where(kh >= cand, ione16, izeros16)
            cnt = _red16(scr_i, c, izeros16, jnp.add)
            return jnp.where(cnt >= kk, cand, t)

        lbk = lax.fori_loop(0, 32, lb_iter, jnp.uint32(0))
        ki = lax.bitcast_convert_type(lbk, jnp.int32)
        lb = lax.bitcast_convert_type(
            ki ^ ((~(ki >> 31)) | jnp.int32(-2147483648)), jnp.float32)

        def b_body(g, off):
            base = g * (GV * L)
            vs = [row_v[pl.ds(base + q * L, L)] for q in range(GV)]
            acc = jnp.maximum(jnp.maximum(vs[0], vs[1]),
                              jnp.maximum(vs[2], vs[3]))
            gm = jnp.where(acc >= lb, ione16, izeros16)
            gbits = _red16(scr_i, gm, izeros16, jnp.bitwise_or)

            def grp():
                ms = [v >= lb for v in vs]
                ws = [jnp.where(m, ione16, izeros16) for m in ms]
                s = ws[0] | (ws[1] << 1) | (ws[2] << 2) | (ws[3] << 3)
                bits = _red16(scr_i, s, izeros16, jnp.bitwise_or)
                o = off
                for q in range(GV):
                    bq = (bits >> q) & 1
                    so = jnp.minimum(o, jnp.int32(CAPW))

                    @pl.when(bq != 0)
                    def _():
                        km = jnp.where(ms[q], _ord_u32(vs[q]), jnp.uint32(0))
                        skey[pl.ds(so, L)] = km
                        sidx[pl.ds(so, L)] = iota16 + (base + q * L)

                    o = o + jnp.where(bq != 0, jnp.int32(L), jnp.int32(0))
                return o

            return lax.cond(gbits != 0, grp, lambda: off)

        off = lax.fori_loop(0, NG, b_body, jnp.int32(0))
        nv = jnp.minimum(off, jnp.int32(CAPW)) // L

        def count_ge(t):
            def cnt(i, acc):
                k = skey[pl.ds(i * L, L)]
                return acc + jnp.where(k >= t, ione16, izeros16)

            acc = lax.fori_loop(0, nv, cnt, izeros16)
            return _red16(scr_i, acc, izeros16, jnp.add)

        def v_iter(i, t):
            cand = t | lax.shift_left(
                jnp.uint32(1), (jnp.int32(31) - i).astype(jnp.uint32))
            return jnp.where(count_ge(cand) >= kk, cand, t)

        thr = lax.fori_loop(0, 32, v_iter, jnp.uint32(0))

        def cnt2(i, acc):
            k = skey[pl.ds(i * L, L)]
            gt = jnp.where(k > thr, ione16, izeros16)
            eq = jnp.where(k == thr, ione16, izeros16)
            return acc + gt + (eq << 8)

        both = lax.fori_loop(0, nv, cnt2, izeros16)
        both_s = _red16(scr_i, both, izeros16, jnp.add)
        cnt_gt = both_s & 0xFF
        t_cnt = both_s >> 8
        m_need = kk - cnt_gt

        def tie_search():
            def tcopy(i, c):
                k = skey[pl.ds(i * L, L)]
                iv = sidx[pl.ds(i * L, L)]
                tbuf[pl.ds(i * L, L)] = jnp.where(k == thr, iv, jnp.int32(BIG))
                return c

            lax.fori_loop(0, nv, tcopy, 0)

            def j_iter(i, jcur):
                cand = jcur | lax.shift_left(jnp.int32(1), jnp.int32(14) - i)

                def cnt(q, acc):
                    iv = tbuf[pl.ds(q * L, L)]
                    return acc + jnp.where(iv < cand, ione16, izeros16)

                acc = lax.fori_loop(0, nv, cnt, izeros16)
                h = _red16(scr_i, acc, izeros16, jnp.add)
                return jnp.where(h < m_need, cand, jcur)

            return lax.fori_loop(0, 15, j_iter, jnp.int32(0))

        jstar = lax.cond(t_cnt > m_need, tie_search, lambda: jnp.int32(N))

        def s_body(i, c):
            k = skey[pl.ds(i * L, L)]
            iv = sidx[pl.ds(i * L, L)]
            ms = jnp.logical_or(
                k > thr, jnp.logical_and(k == thr, iv <= jstar))
            wv = jnp.where(ms, ones16, zeros16)
            bs = iv[0] & jnp.int32(~(L - 1))
            outbuf[pl.ds(bs, L)] = wv
            return c

        lax.fori_loop(0, nv, s_body, 0)

        pltpu.sync_copy(outbuf, out_hbm.at[r])

        def rz(i, c):
            bs = sidx[pl.ds(i * L, L)][0] & jnp.int32(~(L - 1))
            outbuf[pl.ds(bs, L)] = zeros16
            return c

        lax.fori_loop(0, nv, rz, 0)
        return carry

    lax.fori_loop(0, RPW, row_body, 0)


def kernel(logits, shape):
    del shape
    x = logits[..., 0]
    mesh = plsc.VectorSubcoreMesh(core_axis_name="c", subcore_axis_name="s")
    f = functools.partial(
        pl.kernel,
        mesh=mesh,
        out_type=jax.ShapeDtypeStruct((B, N), jnp.float32),
        scratch_types=[
            pltpu.VMEM((N,), jnp.float32),
            pltpu.VMEM((N,), jnp.float32),
            pltpu.VMEM((SCAP,), jnp.uint32),
            pltpu.VMEM((SCAP,), jnp.int32),
            pltpu.VMEM((SCAP,), jnp.int32),
            pltpu.VMEM((2 * L,), jnp.float32),
            pltpu.VMEM((2 * L,), jnp.int32),
        ],
    )(_sc_body)
    out = f(x)
    return out[..., None]


# speedup vs baseline: 1.5453x; 1.0289x over previous
import functools

import jax
import jax.numpy as jnp
from jax import lax
from jax.experimental import pallas as pl
from jax.experimental.pallas import tpu as pltpu
from jax.experimental.pallas import tpu_sc as plsc

K = 32
B = 128
N = 32768
NC = 2
NS = 16
L = 16
NW = NC * NS
RPW = B // NW
NV = N // L
GV = 4
NG = NV // GV
SCAP = 8192
CAPW = SCAP - L
BIG = 2147483647


def _ord_u32(v):
    xb = lax.bitcast_convert_type(v, jnp.int32)
    flip = (xb >> 31) | jnp.int32(-2147483648)
    return lax.bitcast_convert_type(xb ^ flip, jnp.uint32)


def _red16(scr, v, pad, op):
    scr[pl.ds(16, L)] = pad
    s = v
    for sh in (8, 4, 2, 1):
        scr[pl.ds(0, L)] = s
        s = op(s, scr[pl.ds(sh, L)])
    return s[0]


def _sc_body(x_hbm, out_hbm, row_v, outbuf, skey, sidx, tbuf, scr_f, scr_i,
             in_sem, out_sem):
    wid = lax.axis_index("s") * NC + lax.axis_index("c")
    iota16 = lax.broadcasted_iota(jnp.int32, (L,), 0)
    zeros16 = jnp.zeros((L,), jnp.float32)
    ones16 = jnp.ones((L,), jnp.float32)
    izeros16 = jnp.zeros((L,), jnp.int32)
    ione16 = izeros16 + 1
    inf16 = jnp.full((L,), jnp.inf, jnp.float32)
    kk = jnp.int32(K)

    def zinit(i, c):
        outbuf[pl.ds(i * L, L)] = zeros16
        return c

    lax.fori_loop(0, NV, zinit, 0)

    pltpu.async_copy(x_hbm.at[wid * RPW], row_v.at[pl.ds(0, N)], in_sem)

    def row_body(j, prev_off):
        r = wid * RPW + j
        sofs = (j & 1) * N
        pltpu.make_async_copy(
            x_hbm.at[r], row_v.at[pl.ds(sofs, N)], in_sem).wait()

        @pl.when(j + 1 < RPW)
        def _():
            pltpu.async_copy(
                x_hbm.at[r + 1],
                row_v.at[pl.ds(((j + 1) & 1) * N, N)], in_sem)

        @pl.when(j > 0)
        def _():
            pltpu.make_async_copy(outbuf, out_hbm.at[r - 1], out_sem).wait()
            nvp = jnp.minimum(prev_off, jnp.int32(CAPW)) // L

            def rz(i, c):
                bs = sidx[pl.ds(i * L, L)][0] & jnp.int32(~(L - 1))
                outbuf[pl.ds(bs, L)] = zeros16
                return c

            lax.fori_loop(0, nvp, rz, 0)

        def amax(i, h):
            return jnp.maximum(h, row_v[pl.ds(sofs + i * L, L)])

        qn = NV // 4
        khs = []
        for qq in range(4):
            h = lax.fori_loop(qq * qn, (qq + 1) * qn, amax,
                              jnp.full((L,), -jnp.inf))
            khs.append(_ord_u32(h))

        def lb_iter(i, t):
            cand = t | lax.shift_left(
                jnp.uint32(1), (jnp.int32(31) - i).astype(jnp.uint32))
            c = izeros16
            for kh in khs:
                c = c + jnp.where(kh >= cand, ione16, izeros16)
            cnt = _red16(scr_i, c, izeros16, jnp.add)
            return jnp.where(cnt >= kk, cand, t)

        lbk = lax.fori_loop(0, 32, lb_iter, jnp.uint32(0))
        ki = lax.bitcast_convert_type(lbk, jnp.int32)
        lb = lax.bitcast_convert_type(
            ki ^ ((~(ki >> 31)) | jnp.int32(-2147483648)), jnp.float32)

        def b_body(g, off):
            base = g * (GV * L)
            vs = [row_v[pl.ds(base + q * L, L)] for q in range(GV)]
            acc = jnp.maximum(jnp.maximum(vs[0], vs[1]),
                              jnp.maximum(vs[2], vs[3]))
            gm = jnp.where(acc >= lb, ione16, izeros16)
            gbits = _red16(scr_i, gm, izeros16, jnp.bitwise_or)

            def grp():
                ms = [v >= lb for v in vs]
                ws = [jnp.where(m, ione16, izeros16) for m in ms]
                s = ws[0] | (ws[1] << 1) | (ws[2] << 2) | (ws[3] << 3)
                bits = _red16(scr_i, s, izeros16, jnp.bitwise_or)
                o = off
                for q in range(GV):
                    bq = (bits >> q) & 1
                    so = jnp.minimum(o, jnp.int32(CAPW))

                    @pl.when(bq != 0)
                    def _():
                        km = jnp.where(ms[q], _ord_u32(vs[q]), jnp.uint32(0))
                        skey[pl.ds(so, L)] = km
                        sidx[pl.ds(so, L)] = iota16 + (base + q * L)

                    o = o + jnp.where(bq != 0, jnp.int32(L), jnp.int32(0))
                return o

            return lax.cond(gbits != 0, grp, lambda: off)

        off = lax.fori_loop(0, NG, b_body, jnp.int32(0))
        nv = jnp.minimum(off, jnp.int32(CAPW)) // L

        def count_ge(t):
            def cnt(i, acc):
                k = skey[pl.ds(i * L, L)]
                return acc + jnp.where(k >= t, ione16, izeros16)

            acc = lax.fori_loop(0, nv, cnt, izeros16)
            return _red16(scr_i, acc, izeros16, jnp.add)

        def v_iter(i, t):
            cand = t | lax.shift_left(
                jnp.uint32(1), (jnp.int32(31) - i).astype(jnp.uint32))
            return jnp.where(count_ge(cand) >= kk, cand, t)

        thr = lax.fori_loop(0, 32, v_iter, jnp.uint32(0))

        def cnt2(i, acc):
            k = skey[pl.ds(i * L, L)]
            gt = jnp.where(k > thr, ione16, izeros16)
            eq = jnp.where(k == thr, ione16, izeros16)
            return acc + gt + (eq << 8)

        both = lax.fori_loop(0, nv, cnt2, izeros16)
        both_s = _red16(scr_i, both, izeros16, jnp.add)
        cnt_gt = both_s & 0xFF
        t_cnt = both_s >> 8
        m_need = kk - cnt_gt

        def tie_search():
            def tcopy(i, c):
                k = skey[pl.ds(i * L, L)]
                iv = sidx[pl.ds(i * L, L)]
                tbuf[pl.ds(i * L, L)] = jnp.where(k == thr, iv, jnp.int32(BIG))
                return c

            lax.fori_loop(0, nv, tcopy, 0)

            def j_iter(i, jcur):
                cand = jcur | lax.shift_left(jnp.int32(1), jnp.int32(14) - i)

                def cnt(q, acc):
                    iv = tbuf[pl.ds(q * L, L)]
                    return acc + jnp.where(iv < cand, ione16, izeros16)

                acc = lax.fori_loop(0, nv, cnt, izeros16)
                h = _red16(scr_i, acc, izeros16, jnp.add)
                return jnp.where(h < m_need, cand, jcur)

            return lax.fori_loop(0, 15, j_iter, jnp.int32(0))

        jstar = lax.cond(t_cnt > m_need, tie_search, lambda: jnp.int32(N))

        def s_body(i, c):
            k = skey[pl.ds(i * L, L)]
            iv = sidx[pl.ds(i * L, L)]
            ms = jnp.logical_or(
                k > thr, jnp.logical_and(k == thr, iv <= jstar))
            wv = jnp.where(ms, ones16, zeros16)
            bs = iv[0] & jnp.int32(~(L - 1))
            outbuf[pl.ds(bs, L)] = wv
            return c

        lax.fori_loop(0, nv, s_body, 0)

        pltpu.async_copy(outbuf, out_hbm.at[r], out_sem)
        return off

    lax.fori_loop(0, RPW, row_body, jnp.int32(0))
    pltpu.make_async_copy(
        outbuf, out_hbm.at[wid * RPW + RPW - 1], out_sem).wait()


def kernel(logits, shape):
    del shape
    x = logits[..., 0]
    mesh = plsc.VectorSubcoreMesh(core_axis_name="c", subcore_axis_name="s")
    f = functools.partial(
        pl.kernel,
        mesh=mesh,
        out_type=jax.ShapeDtypeStruct((B, N), jnp.float32),
        scratch_types=[
            pltpu.VMEM((2 * N,), jnp.float32),
            pltpu.VMEM((N,), jnp.float32),
            pltpu.VMEM((SCAP,), jnp.uint32),
            pltpu.VMEM((SCAP,), jnp.int32),
            pltpu.VMEM((SCAP,), jnp.int32),
            pltpu.VMEM((2 * L,), jnp.float32),
            pltpu.VMEM((2 * L,), jnp.int32),
            pltpu.SemaphoreType.DMA,
            pltpu.SemaphoreType.DMA,
        ],
    )(_sc_body)
    out = f(x)
    return out[..., None]


# speedup vs baseline: 2.2116x; 1.4312x over previous
import functools

import jax
import jax.numpy as jnp
from jax import lax
from jax.experimental import pallas as pl
from jax.experimental.pallas import tpu as pltpu
from jax.experimental.pallas import tpu_sc as plsc

K = 32
B = 128
N = 32768
NC = 2
NS = 16
L = 16
NW = NC * NS
RPW = B // NW
NV = N // L
GV = 4
NG = NV // GV
SCAP = 8192
CAPW = SCAP - L
BIG = 2147483647


def _ord_u32(v):
    xb = lax.bitcast_convert_type(v, jnp.int32)
    flip = (xb >> 31) | jnp.int32(-2147483648)
    return lax.bitcast_convert_type(xb ^ flip, jnp.uint32)


def _red16(v, op):
    r = op(v, lax.rev(v, (0,)))
    s = r[0]
    for i in range(1, 8):
        s = op(s, r[i])
    return s


def _sc_body(x_hbm, out_hbm, row_v, outbuf, skey, sidx, tbuf):
    wid = lax.axis_index("s") * NC + lax.axis_index("c")
    iota16 = lax.broadcasted_iota(jnp.int32, (L,), 0)
    zeros16 = jnp.zeros((L,), jnp.float32)
    ones16 = jnp.ones((L,), jnp.float32)
    izeros16 = jnp.zeros((L,), jnp.int32)
    ione16 = izeros16 + 1
    inf16 = jnp.full((L,), jnp.inf, jnp.float32)
    kk = jnp.int32(K)

    def zinit(i, c):
        outbuf[pl.ds(i * L, L)] = zeros16
        return c

    lax.fori_loop(0, NV, zinit, 0)

    def row_body(j, carry):
        r = wid * RPW + j
        pltpu.sync_copy(x_hbm.at[r], row_v)

        def amax(i, h):
            return jnp.maximum(h, row_v[pl.ds(i * L, L)])

        qn = NV // 4
        khs = []
        for qq in range(4):
            def amax8(i, h):
                b = i * (8 * L)
                for u in range(8):
                    h = jnp.maximum(h, row_v[pl.ds(b + u * L, L)])
                return h

            h = lax.fori_loop(qq * (qn // 8), (qq + 1) * (qn // 8), amax8,
                              jnp.full((L,), -jnp.inf))
            khs.append(_ord_u32(h))

        def lb_iter(i, t):
            cand = t | lax.shift_left(
                jnp.uint32(1), (jnp.int32(31) - i).astype(jnp.uint32))
            c = izeros16
            for kh in khs:
                c = c + jnp.where(kh >= cand, ione16, izeros16)
            cnt = _red16(c, jnp.add)
            return jnp.where(cnt >= kk, cand, t)

        lbk = lax.fori_loop(0, 32, lb_iter, jnp.uint32(0))
        ki = lax.bitcast_convert_type(lbk, jnp.int32)
        lb = lax.bitcast_convert_type(
            ki ^ ((~(ki >> 31)) | jnp.int32(-2147483648)), jnp.float32)

        def b_body(g, off):
            base = g * (GV * L)
            vs = [row_v[pl.ds(base + q * L, L)] for q in range(GV)]
            acc = jnp.maximum(jnp.maximum(vs[0], vs[1]),
                              jnp.maximum(vs[2], vs[3]))
            gm = jnp.where(acc >= lb, ione16, izeros16)
            gbits = _red16(gm, jnp.bitwise_or)

            def grp():
                ms = [v >= lb for v in vs]
                ws = [jnp.where(m, ione16, izeros16) for m in ms]
                s = ws[0] | (ws[1] << 1) | (ws[2] << 2) | (ws[3] << 3)
                bits = _red16(s, jnp.bitwise_or)
                o = off
                for q in range(GV):
                    bq = (bits >> q) & 1
                    so = jnp.minimum(o, jnp.int32(CAPW))

                    @pl.when(bq != 0)
                    def _():
                        km = jnp.where(ms[q], _ord_u32(vs[q]), jnp.uint32(0))
                        skey[pl.ds(so, L)] = km
                        sidx[pl.ds(so, L)] = iota16 + (base + q * L)

                    o = o + jnp.where(bq != 0, jnp.int32(L), jnp.int32(0))
                return o

            return lax.cond(gbits != 0, grp, lambda: off)

        off = lax.fori_loop(0, NG, b_body, jnp.int32(0))
        offc = jnp.minimum(off, jnp.int32(CAPW))
        nv = offc // L
        nv4 = (nv + 3) // 4
        zk = jnp.zeros((L,), jnp.uint32)
        for u in range(3):
            skey[pl.ds(offc + u * L, L)] = zk

        def count_ge(t):
            def cnt(i, acc):
                b = i * (4 * L)
                for u in range(4):
                    k = skey[pl.ds(b + u * L, L)]
                    acc = acc + jnp.where(k >= t, ione16, izeros16)
                return acc

            acc = lax.fori_loop(0, nv4, cnt, izeros16)
            return _red16(acc, jnp.add)

        def v_iter(i, t):
            cand = t | lax.shift_left(
                jnp.uint32(1), (jnp.int32(31) - i).astype(jnp.uint32))
            return jnp.where(count_ge(cand) >= kk, cand, t)

        thr = lax.fori_loop(0, 32, v_iter, jnp.uint32(0))

        def cnt2(i, acc):
            b = i * (4 * L)
            for u in range(4):
                k = skey[pl.ds(b + u * L, L)]
                gt = jnp.where(k > thr, ione16, izeros16)
                eq = jnp.where(k == thr, ione16, izeros16)
                acc = acc + gt + (eq << 8)
            return acc

        both = lax.fori_loop(0, nv4, cnt2, izeros16)
        both_s = _red16(both, jnp.add)
        cnt_gt = both_s & 0xFF
        t_cnt = both_s >> 8
        m_need = kk - cnt_gt

        def tie_search():
            def tcopy(i, c):
                k = skey[pl.ds(i * L, L)]
                iv = sidx[pl.ds(i * L, L)]
                tbuf[pl.ds(i * L, L)] = jnp.where(k == thr, iv, jnp.int32(BIG))
                return c

            lax.fori_loop(0, nv, tcopy, 0)
            bigv = izeros16 + jnp.int32(BIG)
            for u in range(3):
                tbuf[pl.ds(nv * L + u * L, L)] = bigv

            def j_iter(i, jcur):
                cand = jcur | lax.shift_left(jnp.int32(1), jnp.int32(14) - i)

                def cnt(q, acc):
                    b = q * (4 * L)
                    for u in range(4):
                        iv = tbuf[pl.ds(b + u * L, L)]
                        acc = acc + jnp.where(iv < cand, ione16, izeros16)
                    return acc

                acc = lax.fori_loop(0, nv4, cnt, izeros16)
                h = _red16(acc, jnp.add)
                return jnp.where(h < m_need, cand, jcur)

            return lax.fori_loop(0, 15, j_iter, jnp.int32(0))

        jstar = lax.cond(t_cnt > m_need, tie_search, lambda: jnp.int32(N))

        def s_body(i, c):
            k = skey[pl.ds(i * L, L)]
            iv = sidx[pl.ds(i * L, L)]
            ms = jnp.logical_or(
                k > thr, jnp.logical_and(k == thr, iv <= jstar))
            wv = jnp.where(ms, ones16, zeros16)
            bs = iv[0] & jnp.int32(~(L - 1))
            outbuf[pl.ds(bs, L)] = wv
            return c

        lax.fori_loop(0, nv, s_body, 0)

        pltpu.sync_copy(outbuf, out_hbm.at[r])

        def rz(i, c):
            bs = sidx[pl.ds(i * L, L)][0] & jnp.int32(~(L - 1))
            outbuf[pl.ds(bs, L)] = zeros16
            return c

        lax.fori_loop(0, nv, rz, 0)
        return carry

    lax.fori_loop(0, RPW, row_body, 0)


def kernel(logits, shape):
    del shape
    x = logits[..., 0]
    mesh = plsc.VectorSubcoreMesh(core_axis_name="c", subcore_axis_name="s")
    f = functools.partial(
        pl.kernel,
        mesh=mesh,
        out_type=jax.ShapeDtypeStruct((B, N), jnp.float32),
        scratch_types=[
            pltpu.VMEM((N,), jnp.float32),
            pltpu.VMEM((N,), jnp.float32),
            pltpu.VMEM((SCAP,), jnp.uint32),
            pltpu.VMEM((SCAP,), jnp.int32),
            pltpu.VMEM((SCAP,), jnp.int32),
        ],
    )(_sc_body)
    out = f(x)
    return out[..., None]


# speedup vs baseline: 2.3237x; 1.0507x over previous
import functools

import jax
import jax.numpy as jnp
from jax import lax
from jax.experimental import pallas as pl
from jax.experimental.pallas import tpu as pltpu
from jax.experimental.pallas import tpu_sc as plsc

K = 32
B = 128
N = 32768
NC = 2
NS = 16
L = 16
NW = NC * NS
RPW = B // NW
NV = N // L
GV = 4
NG = NV // GV
SCAP = 8192
CAPW = SCAP - L
BIG = 2147483647


def _ord_u32(v):
    xb = lax.bitcast_convert_type(v, jnp.int32)
    flip = (xb >> 31) | jnp.int32(-2147483648)
    return lax.bitcast_convert_type(xb ^ flip, jnp.uint32)


def _red16(v, op):
    r = op(v, lax.rev(v, (0,)))
    s = r[0]
    for i in range(1, 8):
        s = op(s, r[i])
    return s


def _sc_body(x_hbm, out_hbm, row_v, outbuf, skey, sidx, tbuf):
    wid = lax.axis_index("s") * NC + lax.axis_index("c")
    iota16 = lax.broadcasted_iota(jnp.int32, (L,), 0)
    zeros16 = jnp.zeros((L,), jnp.float32)
    ones16 = jnp.ones((L,), jnp.float32)
    izeros16 = jnp.zeros((L,), jnp.int32)
    ione16 = izeros16 + 1
    inf16 = jnp.full((L,), jnp.inf, jnp.float32)
    kk = jnp.int32(K)

    def zinit(i, c):
        b = i * (8 * L)
        for u in range(8):
            outbuf[pl.ds(b + u * L, L)] = zeros16
        return c

    lax.fori_loop(0, NV // 8, zinit, 0)

    def row_body(j, carry):
        r = wid * RPW + j
        pltpu.sync_copy(x_hbm.at[r], row_v)

        def amax(i, h):
            return jnp.maximum(h, row_v[pl.ds(i * L, L)])

        qn = NV // 4
        khs = []
        for qq in range(4):
            def amax8(i, h):
                b = i * (8 * L)
                for u in range(8):
                    h = jnp.maximum(h, row_v[pl.ds(b + u * L, L)])
                return h

            h = lax.fori_loop(qq * (qn // 8), (qq + 1) * (qn // 8), amax8,
                              jnp.full((L,), -jnp.inf))
            khs.append(_ord_u32(h))

        def lb_iter(i, t):
            cand = t | lax.shift_left(
                jnp.uint32(1), (jnp.int32(31) - i).astype(jnp.uint32))
            c = izeros16
            for kh in khs:
                c = c + jnp.where(kh >= cand, ione16, izeros16)
            cnt = _red16(c, jnp.add)
            return jnp.where(cnt >= kk, cand, t)

        lbk = lax.fori_loop(0, 32, lb_iter, jnp.uint32(0))
        ki = lax.bitcast_convert_type(lbk, jnp.int32)
        lb = lax.bitcast_convert_type(
            ki ^ ((~(ki >> 31)) | jnp.int32(-2147483648)), jnp.float32)

        def b_body(g, off):
            base = g * (GV * L)
            vs = [row_v[pl.ds(base + q * L, L)] for q in range(GV)]
            acc = jnp.maximum(jnp.maximum(vs[0], vs[1]),
                              jnp.maximum(vs[2], vs[3]))
            gm = jnp.where(acc >= lb, ione16, izeros16)
            gbits = _red16(gm, jnp.bitwise_or)

            def grp():
                ms = [v >= lb for v in vs]
                ws = [jnp.where(m, ione16, izeros16) for m in ms]
                s = ws[0] | (ws[1] << 1) | (ws[2] << 2) | (ws[3] << 3)
                bits = _red16(s, jnp.bitwise_or)
                o = off
                for q in range(GV):
                    bq = (bits >> q) & 1
                    so = jnp.minimum(o, jnp.int32(CAPW))

                    @pl.when(bq != 0)
                    def _():
                        km = jnp.where(ms[q], _ord_u32(vs[q]), jnp.uint32(0))
                        skey[pl.ds(so, L)] = km
                        sidx[pl.ds(so, L)] = iota16 + (base + q * L)

                    o = o + jnp.where(bq != 0, jnp.int32(L), jnp.int32(0))
                return o

            return lax.cond(gbits != 0, grp, lambda: off)

        off = lax.fori_loop(0, NG, b_body, jnp.int32(0))
        offc = jnp.minimum(off, jnp.int32(CAPW))
        nv = offc // L
        nv4 = (nv + 3) // 4
        zk = jnp.zeros((L,), jnp.uint32)
        for u in range(3):
            skey[pl.ds(offc + u * L, L)] = zk

        def count_ge(t):
            def cnt(i, acc):
                b = i * (4 * L)
                for u in range(4):
                    k = skey[pl.ds(b + u * L, L)]
                    acc = acc + jnp.where(k >= t, ione16, izeros16)
                return acc

            acc = lax.fori_loop(0, nv4, cnt, izeros16)
            return _red16(acc, jnp.add)

        def v_iter(i, t):
            cand = t | lax.shift_left(
                jnp.uint32(1), (jnp.int32(31) - i).astype(jnp.uint32))
            return jnp.where(count_ge(cand) >= kk, cand, t)

        thr = lax.fori_loop(0, 32, v_iter, jnp.uint32(0))

        def cnt2(i, acc):
            b = i * (4 * L)
            for u in range(4):
                k = skey[pl.ds(b + u * L, L)]
                gt = jnp.where(k > thr, ione16, izeros16)
                eq = jnp.where(k == thr, ione16, izeros16)
                acc = acc + gt + (eq << 8)
            return acc

        both = lax.fori_loop(0, nv4, cnt2, izeros16)
        both_s = _red16(both, jnp.add)
        cnt_gt = both_s & 0xFF
        t_cnt = both_s >> 8
        m_need = kk - cnt_gt

        def tie_search():
            def tcopy(i, c):
                k = skey[pl.ds(i * L, L)]
                iv = sidx[pl.ds(i * L, L)]
                tbuf[pl.ds(i * L, L)] = jnp.where(k == thr, iv, jnp.int32(BIG))
                return c

            lax.fori_loop(0, nv, tcopy, 0)
            bigv = izeros16 + jnp.int32(BIG)
            for u in range(3):
                tbuf[pl.ds(nv * L + u * L, L)] = bigv

            def j_iter(i, jcur):
                cand = jcur | lax.shift_left(jnp.int32(1), jnp.int32(14) - i)

                def cnt(q, acc):
                    b = q * (4 * L)
                    for u in range(4):
                        iv = tbuf[pl.ds(b + u * L, L)]
                        acc = acc + jnp.where(iv < cand, ione16, izeros16)
                    return acc

                acc = lax.fori_loop(0, nv4, cnt, izeros16)
                h = _red16(acc, jnp.add)
                return jnp.where(h < m_need, cand, jcur)

            return lax.fori_loop(0, 15, j_iter, jnp.int32(0))

        jstar = lax.cond(t_cnt > m_need, tie_search, lambda: jnp.int32(N))

        def s_body(i, c):
            k = skey[pl.ds(i * L, L)]
            iv = sidx[pl.ds(i * L, L)]
            ms = jnp.logical_or(
                k > thr, jnp.logical_and(k == thr, iv <= jstar))
            wv = jnp.where(ms, ones16, zeros16)
            bs = iv[0] & jnp.int32(~(L - 1))
            outbuf[pl.ds(bs, L)] = wv
            return c

        lax.fori_loop(0, nv, s_body, 0)

        pltpu.sync_copy(outbuf, out_hbm.at[r])

        def rz(i, c):
            bs = sidx[pl.ds(i * L, L)][0] & jnp.int32(~(L - 1))
            outbuf[pl.ds(bs, L)] = zeros16
            return c

        lax.fori_loop(0, nv, rz, 0)
        return carry

    lax.fori_loop(0, RPW, row_body, 0)


def kernel(logits, shape):
    del shape
    x = logits[..., 0]
    mesh = plsc.VectorSubcoreMesh(core_axis_name="c", subcore_axis_name="s")
    f = functools.partial(
        pl.kernel,
        mesh=mesh,
        out_type=jax.ShapeDtypeStruct((B, N), jnp.float32),
        scratch_types=[
            pltpu.VMEM((N,), jnp.float32),
            pltpu.VMEM((N,), jnp.float32),
            pltpu.VMEM((SCAP,), jnp.uint32),
            pltpu.VMEM((SCAP,), jnp.int32),
            pltpu.VMEM((SCAP,), jnp.int32),
        ],
    )(_sc_body)
    out = f(x)
    return out[..., None]


# speedup vs baseline: 3.1991x; 1.3767x over previous
import functools

import jax
import jax.numpy as jnp
from jax import lax
from jax.experimental import pallas as pl
from jax.experimental.pallas import tpu as pltpu
from jax.experimental.pallas import tpu_sc as plsc

K = 32
B = 128
N = 32768
NC = 2
NS = 16
L = 16
NW = NC * NS
RPW = B // NW
NV = N // L
GV = 4
NG = NV // GV
SCAP = 8192
CAPW = SCAP - L
BIG = 2147483647


def _ord_u32(v):
    xb = lax.bitcast_convert_type(v, jnp.int32)
    flip = (xb >> 31) | jnp.int32(-2147483648)
    return lax.bitcast_convert_type(xb ^ flip, jnp.uint32)


def _red16(v, op):
    r = op(v, lax.rev(v, (0,)))
    s = r[0]
    for i in range(1, 8):
        s = op(s, r[i])
    return s


def _sc_body(x_hbm, out_hbm, row_v, outbuf, skey, sidx, tbuf):
    wid = lax.axis_index("s") * NC + lax.axis_index("c")
    iota16 = lax.broadcasted_iota(jnp.int32, (L,), 0)
    zeros16 = jnp.zeros((L,), jnp.float32)
    ones16 = jnp.ones((L,), jnp.float32)
    izeros16 = jnp.zeros((L,), jnp.int32)
    ione16 = izeros16 + 1
    inf16 = jnp.full((L,), jnp.inf, jnp.float32)
    kk = jnp.int32(K)

    def zinit(i, c):
        b = i * (8 * L)
        for u in range(8):
            outbuf[pl.ds(b + u * L, L)] = zeros16
        return c

    lax.fori_loop(0, NV // 8, zinit, 0)

    def row_body(j, carry):
        r = wid * RPW + j
        pltpu.sync_copy(x_hbm.at[r], row_v)

        def amax(i, h):
            return jnp.maximum(h, row_v[pl.ds(i * L, L)])

        qn = NV // 4
        khs = []
        for qq in range(4):
            def amax8(i, h):
                b = i * (8 * L)
                for u in range(8):
                    h = jnp.maximum(h, row_v[pl.ds(b + u * L, L)])
                return h

            h = lax.fori_loop(qq * (qn // 8), (qq + 1) * (qn // 8), amax8,
                              jnp.full((L,), -jnp.inf))
            khs.append(_ord_u32(h))

        def lb_iter(i, t):
            cand = t | lax.shift_left(
                jnp.uint32(1), (jnp.int32(31) - i).astype(jnp.uint32))
            c = izeros16
            for kh in khs:
                c = c + jnp.where(kh >= cand, ione16, izeros16)
            cnt = _red16(c, jnp.add)
            return jnp.where(cnt >= kk, cand, t)

        lbk = lax.fori_loop(0, 32, lb_iter, jnp.uint32(0))
        ki = lax.bitcast_convert_type(lbk, jnp.int32)
        lb = lax.bitcast_convert_type(
            ki ^ ((~(ki >> 31)) | jnp.int32(-2147483648)), jnp.float32)

        bitvs = [izeros16 + (1 << t) for t in range(16)]

        def b_pack(p, off):
            s = izeros16
            for t in range(16):
                base = (p * 16 + t) * (GV * L)
                acc = row_v[pl.ds(base, L)]
                for q in range(1, GV):
                    acc = jnp.maximum(acc, row_v[pl.ds(base + q * L, L)])
                s = s | jnp.where(acc >= lb, bitvs[t], izeros16)
            bits = _red16(s, jnp.bitwise_or)

            o = off
            for t in range(16):
                gb = (bits >> t) & 1
                base_t = (p * 16 + t) * (GV * L)

                def mk(base_c, oo):
                    def grp():
                        vs = [row_v[pl.ds(base_c + q * L, L)]
                              for q in range(GV)]
                        ms = [v >= lb for v in vs]
                        ws = [jnp.where(m, ione16, izeros16) for m in ms]
                        sv = ws[0] | (ws[1] << 1) | (ws[2] << 2) | (ws[3] << 3)
                        b2 = _red16(sv, jnp.bitwise_or)
                        o2 = oo
                        for q in range(GV):
                            bq = (b2 >> q) & 1
                            so = jnp.minimum(o2, jnp.int32(CAPW))

                            @pl.when(bq != 0)
                            def _(q=q, so=so):
                                km = jnp.where(ms[q], _ord_u32(vs[q]),
                                               jnp.uint32(0))
                                skey[pl.ds(so, L)] = km
                                sidx[pl.ds(so, L)] = iota16 + (base_c + q * L)

                            o2 = o2 + jnp.where(bq != 0, jnp.int32(L),
                                                jnp.int32(0))
                        return o2
                    return grp

                o = lax.cond(gb != 0, mk(base_t, o), lambda o=o: o)
            return o

        off = lax.fori_loop(0, NG // 16, b_pack, jnp.int32(0))
        offc = jnp.minimum(off, jnp.int32(CAPW))
        nv = offc // L
        nv4 = (nv + 3) // 4
        zk = jnp.zeros((L,), jnp.uint32)
        for u in range(3):
            skey[pl.ds(offc + u * L, L)] = zk

        def count_ge(t):
            def cnt(i, acc):
                b = i * (4 * L)
                for u in range(4):
                    k = skey[pl.ds(b + u * L, L)]
                    acc = acc + jnp.where(k >= t, ione16, izeros16)
                return acc

            acc = lax.fori_loop(0, nv4, cnt, izeros16)
            return _red16(acc, jnp.add)

        def v_iter(i, t):
            cand = t | lax.shift_left(
                jnp.uint32(1), (jnp.int32(31) - i).astype(jnp.uint32))
            return jnp.where(count_ge(cand) >= kk, cand, t)

        thr = lax.fori_loop(0, 32, v_iter, jnp.uint32(0))

        def cnt2(i, acc):
            b = i * (4 * L)
            for u in range(4):
                k = skey[pl.ds(b + u * L, L)]
                gt = jnp.where(k > thr, ione16, izeros16)
                eq = jnp.where(k == thr, ione16, izeros16)
                acc = acc + gt + (eq << 8)
            return acc

        both = lax.fori_loop(0, nv4, cnt2, izeros16)
        both_s = _red16(both, jnp.add)
        cnt_gt = both_s & 0xFF
        t_cnt = both_s >> 8
        m_need = kk - cnt_gt

        def tie_search():
            def tcopy(i, c):
                k = skey[pl.ds(i * L, L)]
                iv = sidx[pl.ds(i * L, L)]
                tbuf[pl.ds(i * L, L)] = jnp.where(k == thr, iv, jnp.int32(BIG))
                return c

            lax.fori_loop(0, nv, tcopy, 0)
            bigv = izeros16 + jnp.int32(BIG)
            for u in range(3):
                tbuf[pl.ds(nv * L + u * L, L)] = bigv

            def j_iter(i, jcur):
                cand = jcur | lax.shift_left(jnp.int32(1), jnp.int32(14) - i)

                def cnt(q, acc):
                    b = q * (4 * L)
                    for u in range(4):
                        iv = tbuf[pl.ds(b + u * L, L)]
                        acc = acc + jnp.where(iv < cand, ione16, izeros16)
                    return acc

                acc = lax.fori_loop(0, nv4, cnt, izeros16)
                h = _red16(acc, jnp.add)
                return jnp.where(h < m_need, cand, jcur)

            return lax.fori_loop(0, 15, j_iter, jnp.int32(0))

        jstar = lax.cond(t_cnt > m_need, tie_search, lambda: jnp.int32(N))

        def s_body(i, c):
            k = skey[pl.ds(i * L, L)]
            iv = sidx[pl.ds(i * L, L)]
            ms = jnp.logical_or(
                k > thr, jnp.logical_and(k == thr, iv <= jstar))
            wv = jnp.where(ms, ones16, zeros16)
            bs = iv[0] & jnp.int32(~(L - 1))
            outbuf[pl.ds(bs, L)] = wv
            return c

        lax.fori_loop(0, nv, s_body, 0)

        pltpu.sync_copy(outbuf, out_hbm.at[r])

        def rz(i, c):
            bs = sidx[pl.ds(i * L, L)][0] & jnp.int32(~(L - 1))
            outbuf[pl.ds(bs, L)] = zeros16
            return c

        lax.fori_loop(0, nv, rz, 0)
        return carry

    lax.fori_loop(0, RPW, row_body, 0)


def kernel(logits, shape):
    del shape
    x = logits[..., 0]
    mesh = plsc.VectorSubcoreMesh(core_axis_name="c", subcore_axis_name="s")
    f = functools.partial(
        pl.kernel,
        mesh=mesh,
        out_type=jax.ShapeDtypeStruct((B, N), jnp.float32),
        scratch_types=[
            pltpu.VMEM((N,), jnp.float32),
            pltpu.VMEM((N,), jnp.float32),
            pltpu.VMEM((SCAP,), jnp.uint32),
            pltpu.VMEM((SCAP,), jnp.int32),
            pltpu.VMEM((SCAP,), jnp.int32),
        ],
    )(_sc_body)
    out = f(x)
    return out[..., None]
